# baseline (speedup 1.0000x reference)
import functools

import jax
import jax.numpy as jnp
from jax import lax
from jax.experimental import pallas as pl
from jax.experimental.pallas import tpu as pltpu
from jax.experimental.pallas import tpu_sc as plsc

_TOPK = 32
_NC = 8


def _proj_body(q_ref, wq_ref, qm_ref):
    qb = q_ref[0].astype(jnp.bfloat16)
    wqb = wq_ref[...].astype(jnp.bfloat16)
    q = lax.dot_general(qb, wqb, (((1,), (1,)), ((), ())),
                        preferred_element_type=jnp.float32)
    s_len = q.shape[0]
    qm_ref[0] = jnp.sum(q, axis=0, keepdims=True) * (1.0 / s_len)


def _scores_topk_body(qm_ref, wk_ref, keys_ref, idx_ref, w_ref, s_scr, *,
                      m_total, scale):
    b = pl.program_id(0)
    j = pl.program_id(1)
    keys = keys_ref[0].astype(jnp.bfloat16)
    wkb = wk_ref[...].astype(jnp.bfloat16)
    k = lax.dot_general(keys, wkb, (((1,), (1,)), ((), ())),
                        preferred_element_type=jnp.float32)
    qmb = qm_ref[0].astype(jnp.bfloat16)
    kb = k.astype(jnp.bfloat16)
    s = lax.dot_general(qmb, kb, (((1,), (1,)), ((), ())),
                        preferred_element_type=jnp.float32)[0] * scale
    rows = s.shape[0] // 128
    s_scr[pl.ds(pl.multiple_of(j * rows, 8), rows), :] = s.reshape(rows, 128)

    @pl.when(j == pl.num_programs(1) - 1)
    def _():
        full = s_scr[...]
        r_tot, c_tot = full.shape
        flat = (lax.broadcasted_iota(jnp.int32, (r_tot, c_tot), 0) * c_tot
                + lax.broadcasted_iota(jnp.int32, (r_tot, c_tot), 1))
        lane = lax.broadcasted_iota(jnp.int32, (1, 128), 1)
        neg = jnp.float32(-jnp.inf)

        def body(i, carry):
            sc, tv, ti = carry
            m = jnp.max(sc)
            idx = jnp.min(jnp.where(sc == m, flat, jnp.int32(2 ** 30)))
            tv = jnp.where(lane == i, m, tv)
            ti = jnp.where(lane == i, idx, ti)
            sc = jnp.where(flat == idx, neg, sc)
            return sc, tv, ti

        tv0 = jnp.full((1, 128), neg, jnp.float32)
        ti0 = jnp.zeros((1, 128), jnp.int32)
        _, tv, ti = lax.fori_loop(0, _TOPK, body, (full, tv0, ti0))

        valid = lane < _TOPK
        mx = jnp.max(jnp.where(valid, tv, neg))
        e = jnp.where(valid, jnp.exp(tv - mx), jnp.float32(0.0))
        w = e / jnp.sum(e)
        idx_ref[0] = ti + b * m_total
        w_ref[0] = w


@functools.lru_cache(maxsize=None)
def _make_sc_gather(n_rows, h):
    n_workers = 16
    rpw = n_rows // n_workers
    mesh = plsc.VectorSubcoreMesh(core_axis_name="c", subcore_axis_name="s")

    @functools.partial(
        pl.kernel, mesh=mesh,
        out_type=jax.ShapeDtypeStruct((n_rows, h), jnp.float32),
        scratch_types=[
            pltpu.VMEM((rpw,), jnp.int32),
            pltpu.VMEM((rpw, h), jnp.float32),
            pltpu.VMEM((rpw, 16), jnp.float32),
            pltpu.SemaphoreType.DMA,
        ],
    )
    def gather_k(values_hbm, idx_hbm, wrep_hbm, out_hbm, idx_v, rows_v, w_v, sem):
        wid = lax.axis_index("s") * 2 + lax.axis_index("c")

        @pl.when(wid < n_workers)
        def _():
            base = wid * rpw
            pltpu.sync_copy(idx_hbm.at[pl.ds(base, rpw)], idx_v)
            pltpu.sync_copy(wrep_hbm.at[pl.ds(base, rpw)], w_v)
            pltpu.async_copy(values_hbm.at[idx_v], rows_v, sem).wait()
            for r in range(rpw):
                wv = w_v[r, :]

                def mul_body(c, carry, r=r, wv=wv):
                    off = c * 16
                    rows_v[r, pl.ds(off, 16)] = rows_v[r, pl.ds(off, 16)] * wv
                    return carry

                lax.fori_loop(0, h // 16, mul_body, 0)
            pltpu.sync_copy(rows_v, out_hbm.at[pl.ds(base, rpw)])

    return gather_k


def kernel(query, memory_keys, memory_values, Wq, Wk):
    B, S, H = query.shape
    M = memory_keys.shape[1]
    KD = Wq.shape[0]
    scale = KD ** (-0.5)
    ch = M // _NC

    qm = pl.pallas_call(
        _proj_body,
        grid=(B,),
        in_specs=[
            pl.BlockSpec((1, S, H), lambda b: (b, 0, 0)),
            pl.BlockSpec((KD, H), lambda b: (0, 0)),
        ],
        out_specs=pl.BlockSpec((1, 1, KD), lambda b: (b, 0, 0)),
        out_shape=jax.ShapeDtypeStruct((B, 1, KD), jnp.float32),
    )(query, Wq)

    idx_pad, w_pad = pl.pallas_call(
        functools.partial(_scores_topk_body, m_total=M, scale=scale),
        grid=(B, _NC),
        in_specs=[
            pl.BlockSpec((1, 1, KD), lambda b, j: (b, 0, 0)),
            pl.BlockSpec((KD, H), lambda b, j: (0, 0)),
            pl.BlockSpec((1, ch, H), lambda b, j: (b, j, 0)),
        ],
        out_specs=[
            pl.BlockSpec((1, 1, 128), lambda b, j: (b, 0, 0)),
            pl.BlockSpec((1, 1, 128), lambda b, j: (b, 0, 0)),
        ],
        out_shape=[
            jax.ShapeDtypeStruct((B, 1, 128), jnp.int32),
            jax.ShapeDtypeStruct((B, 1, 128), jnp.float32),
        ],
        scratch_shapes=[pltpu.VMEM((M // 128, 128), jnp.float32)],
    )(qm, Wk, memory_keys)

    idx_flat = idx_pad[:, 0, :_TOPK].reshape(B * _TOPK)
    w_flat = w_pad[:, 0, :_TOPK].reshape(B * _TOPK)
    wrep = jnp.broadcast_to(w_flat[:, None], (B * _TOPK, 16))
    values_flat = memory_values.reshape(B * M, H)

    out = _make_sc_gather(B * _TOPK, H)(values_flat, idx_flat, wrep)
    return out.reshape(B, _TOPK, H)
